# baseline (speedup 1.0000x reference)
import functools

import jax
import jax.numpy as jnp
from jax import lax
from jax.experimental import pallas as pl
from jax.experimental.pallas import tpu as pltpu
from jax.experimental.pallas import tpu_sc as plsc

N = 10000
E = 320000
D = 128

NC = 2
NS = 16
NT = NC * NS
CHUNK = 128
CH = 80
RING = 40
EPT = CH * CHUNK
EPAD = NT * EPT
NACC = 10240
RPT = NACC // NS


def _sc_segment_sum(h, srcs, dsts, zeros):
    mesh = plsc.VectorSubcoreMesh(core_axis_name="c", subcore_axis_name="s")

    @functools.partial(
        pl.kernel,
        out_type=jax.ShapeDtypeStruct((NC, NACC, D), jnp.float32),
        mesh=mesh,
        scratch_types=[
            pltpu.VMEM((RING, CHUNK), jnp.int32),
            pltpu.VMEM((RING, CHUNK), jnp.int32),
            pltpu.VMEM((CHUNK, D), jnp.float32),
            pltpu.VMEM((CHUNK, D), jnp.float32),
            pltpu.VMEM_SHARED((NACC, D), jnp.float32),
            pltpu.SemaphoreType.DMA,
            pltpu.SemaphoreType.DMA,
        ],
    )
    def seg_sum(h_hbm, srcs_hbm, dsts_hbm, zeros_hbm, out_hbm,
                src_v, dst_v, rows0, rows1, acc, sem0, sem1):
        c = lax.axis_index("c")
        s = lax.axis_index("s")
        wid = c * NS + s
        pltpu.sync_copy(zeros_hbm, acc.at[pl.ds(s * RPT, RPT)])
        plsc.subcore_barrier()

        for r in range(CH // RING):
            pltpu.sync_copy(srcs_hbm.at[wid, pl.ds(r * RING, RING)], src_v)
            pltpu.sync_copy(dsts_hbm.at[wid, pl.ds(r * RING, RING)], dst_v)
            pltpu.async_copy(h_hbm.at[src_v.at[0]], rows0, sem0)

            @pl.loop(0, RING, step=2)
            def chunk(j):
                pltpu.make_async_copy(
                    h_hbm.at[src_v.at[j]], rows0, sem0).wait()
                pltpu.async_copy(h_hbm.at[src_v.at[j + 1]], rows1, sem1)
                pltpu.sync_copy(rows0, acc.at[dst_v.at[j]], add=True)
                pltpu.make_async_copy(
                    h_hbm.at[src_v.at[j + 1]], rows1, sem1).wait()

                @pl.when(j + 2 < RING)
                def _():
                    pltpu.async_copy(h_hbm.at[src_v.at[j + 2]], rows0, sem0)

                pltpu.sync_copy(rows1, acc.at[dst_v.at[j + 1]], add=True)

        plsc.subcore_barrier()
        pltpu.sync_copy(acc.at[pl.ds(s * RPT, RPT)],
                        out_hbm.at[c, pl.ds(s * RPT, RPT)])

    return seg_sum(h, srcs, dsts, zeros)


BM = 1000


def _mlp_body(eps_ref, h_ref, agg_ref, w1_ref, b1_ref, w2_ref, b2_ref, o_ref):
    m = (1.0 + eps_ref[0, 0]) * h_ref[...] + agg_ref[0] + agg_ref[1]
    t = jnp.dot(m, w1_ref[...], preferred_element_type=jnp.float32)
    t = jnp.maximum(t + b1_ref[...], 0.0)
    u = jnp.dot(t, w2_ref[...], preferred_element_type=jnp.float32)
    o_ref[...] = jnp.maximum(u + b2_ref[...], 0.0)


def _tc_gin_mlp(h, aggs, eps, W1, b1, W2, b2):
    return pl.pallas_call(
        _mlp_body,
        grid=(N // BM,),
        in_specs=[
            pl.BlockSpec(memory_space=pltpu.SMEM),
            pl.BlockSpec((BM, D), lambda i: (i, 0)),
            pl.BlockSpec((NC, BM, D), lambda i: (0, i, 0)),
            pl.BlockSpec((D, D), lambda i: (0, 0)),
            pl.BlockSpec((1, D), lambda i: (0, 0)),
            pl.BlockSpec((D, D), lambda i: (0, 0)),
            pl.BlockSpec((1, D), lambda i: (0, 0)),
        ],
        out_specs=pl.BlockSpec((BM, D), lambda i: (i, 0)),
        out_shape=jax.ShapeDtypeStruct((N, D), jnp.float32),
    )(eps.reshape(1, 1), h, aggs, W1, b1.reshape(1, D), W2, b2.reshape(1, D))


def _head_body(eps_ref, fcb_ref, h1_ref, agg_ref, w1_ref, b1_ref, w2_ref,
               b2_ref, fca_ref, fcc_ref, y_ref):
    h1 = h1_ref[...]
    m = (1.0 + eps_ref[0, 0]) * h1 + agg_ref[0] + agg_ref[1]
    t = jnp.dot(m, w1_ref[...], preferred_element_type=jnp.float32)
    t = jnp.maximum(t + b1_ref[...], 0.0)
    u = jnp.dot(t, w2_ref[...], preferred_element_type=jnp.float32)
    h2 = jnp.maximum(u + b2_ref[...], 0.0)
    y = jnp.dot(h1, fca_ref[...], preferred_element_type=jnp.float32)
    y = y + jnp.dot(h2, fcc_ref[...], preferred_element_type=jnp.float32)
    y_ref[...] = y + fcb_ref[0, 0]


def _tc_gin_head(h1, aggs, eps, W1, b1, W2, b2, fc_W, fc_b):
    return pl.pallas_call(
        _head_body,
        grid=(N // BM,),
        in_specs=[
            pl.BlockSpec(memory_space=pltpu.SMEM),
            pl.BlockSpec(memory_space=pltpu.SMEM),
            pl.BlockSpec((BM, D), lambda i: (i, 0)),
            pl.BlockSpec((NC, BM, D), lambda i: (0, i, 0)),
            pl.BlockSpec((D, D), lambda i: (0, 0)),
            pl.BlockSpec((1, D), lambda i: (0, 0)),
            pl.BlockSpec((D, D), lambda i: (0, 0)),
            pl.BlockSpec((1, D), lambda i: (0, 0)),
            pl.BlockSpec((D, 1), lambda i: (0, 0)),
            pl.BlockSpec((D, 1), lambda i: (0, 0)),
        ],
        out_specs=pl.BlockSpec((BM, 1), lambda i: (i, 0)),
        out_shape=jax.ShapeDtypeStruct((N, 1), jnp.float32),
    )(eps.reshape(1, 1), fc_b.reshape(1, 1), h1, aggs, W1, b1.reshape(1, D),
      W2, b2.reshape(1, D), fc_W[:D], fc_W[D:])


def kernel(x, edge_index, ano_label, W1_0, b1_0, W2_0, b2_0, eps_0,
           W1_1, b1_1, W2_1, b2_1, eps_1, fc_W, fc_b):
    del ano_label
    src = edge_index[0].astype(jnp.int32)
    dst = edge_index[1].astype(jnp.int32)
    pad = EPAD - E
    src_p = jnp.concatenate([src, jnp.zeros((pad,), jnp.int32)])
    dst_p = jnp.concatenate([dst, jnp.full((pad,), N, jnp.int32)])
    srcs = src_p.reshape(NT, CH, CHUNK)
    dsts = dst_p.reshape(NT, CH, CHUNK)
    zeros = jnp.zeros((RPT, D), jnp.float32)

    aggs0 = _sc_segment_sum(x, srcs, dsts, zeros)
    h1 = _tc_gin_mlp(x, aggs0, eps_0, W1_0, b1_0, W2_0, b2_0)
    aggs1 = _sc_segment_sum(h1, srcs, dsts, zeros)
    return _tc_gin_head(h1, aggs1, eps_1, W1_1, b1_1, W2_1, b2_1, fc_W, fc_b)


# speedup vs baseline: 1.9964x; 1.9964x over previous
import functools

import jax
import jax.numpy as jnp
from jax import lax
from jax.experimental import pallas as pl
from jax.experimental.pallas import tpu as pltpu
from jax.experimental.pallas import tpu_sc as plsc

N = 10000
E = 320000
D = 128

NC = 2
NS = 16
NT = NC * NS
CHUNK = 128
CH = 79
EPT = CH * CHUNK
EPAD = NT * EPT
NACC = 10240
RPT = NACC // NS


def _sc_segment_sum(h, srcs, dsts, zeros):
    mesh = plsc.VectorSubcoreMesh(core_axis_name="c", subcore_axis_name="s")

    @functools.partial(
        pl.kernel,
        out_type=jax.ShapeDtypeStruct((NC, NACC, D), jnp.bfloat16),
        mesh=mesh,
        scratch_types=[
            pltpu.VMEM((CH, CHUNK), jnp.int32),
            pltpu.VMEM((CH, CHUNK), jnp.int32),
            pltpu.VMEM((CHUNK, D), jnp.bfloat16),
            pltpu.VMEM_SHARED((NACC, D), jnp.bfloat16),
            pltpu.SemaphoreType.DMA,
        ],
        compiler_params=pltpu.CompilerParams(use_tc_tiling_on_sc=False),
    )
    def seg_sum(h_hbm, srcs_hbm, dsts_hbm, zeros_hbm, out_hbm,
                src_v, dst_v, rows_v, acc, sem):
        c = lax.axis_index("c")
        s = lax.axis_index("s")
        wid = c * NS + s
        pltpu.sync_copy(zeros_hbm, acc.at[pl.ds(s * RPT, RPT)])
        pltpu.sync_copy(srcs_hbm.at[wid], src_v)
        pltpu.sync_copy(dsts_hbm.at[wid], dst_v)
        plsc.subcore_barrier()

        @pl.loop(0, CH)
        def chunk(j):
            pltpu.async_copy(h_hbm.at[src_v.at[j]], rows_v, sem).wait()
            pltpu.sync_copy(rows_v, acc.at[dst_v.at[j]], add=True)

        plsc.subcore_barrier()
        pltpu.sync_copy(acc.at[pl.ds(s * RPT, RPT)],
                        out_hbm.at[c, pl.ds(s * RPT, RPT)])

    return seg_sum(h, srcs, dsts, zeros)


BM = 1000


def _mlp_body(eps_ref, h_ref, agg_ref, w1_ref, b1_ref, w2_ref, b2_ref,
              o_ref, obf_ref):
    agg = (agg_ref[0] + agg_ref[1]).astype(jnp.float32)
    m = (1.0 + eps_ref[0, 0]) * h_ref[...] + agg
    t = jnp.dot(m, w1_ref[...], preferred_element_type=jnp.float32)
    t = jnp.maximum(t + b1_ref[...], 0.0)
    u = jnp.dot(t, w2_ref[...], preferred_element_type=jnp.float32)
    h = jnp.maximum(u + b2_ref[...], 0.0)
    o_ref[...] = h
    obf_ref[...] = h.astype(jnp.bfloat16)


def _tc_gin_mlp(h, aggs, eps, W1, b1, W2, b2):
    return pl.pallas_call(
        _mlp_body,
        grid=(N // BM,),
        in_specs=[
            pl.BlockSpec(memory_space=pltpu.SMEM),
            pl.BlockSpec((BM, D), lambda i: (i, 0)),
            pl.BlockSpec((NC, BM, D), lambda i: (0, i, 0)),
            pl.BlockSpec((D, D), lambda i: (0, 0)),
            pl.BlockSpec((1, D), lambda i: (0, 0)),
            pl.BlockSpec((D, D), lambda i: (0, 0)),
            pl.BlockSpec((1, D), lambda i: (0, 0)),
        ],
        out_specs=[
            pl.BlockSpec((BM, D), lambda i: (i, 0)),
            pl.BlockSpec((BM, D), lambda i: (i, 0)),
        ],
        out_shape=[
            jax.ShapeDtypeStruct((N, D), jnp.float32),
            jax.ShapeDtypeStruct((N, D), jnp.bfloat16),
        ],
    )(eps.reshape(1, 1), h, aggs, W1, b1.reshape(1, D), W2, b2.reshape(1, D))


def _head_body(eps_ref, fcb_ref, h1_ref, agg_ref, w1_ref, b1_ref, w2_ref,
               b2_ref, fca_ref, fcc_ref, y_ref):
    h1 = h1_ref[...]
    agg = (agg_ref[0] + agg_ref[1]).astype(jnp.float32)
    m = (1.0 + eps_ref[0, 0]) * h1 + agg
    t = jnp.dot(m, w1_ref[...], preferred_element_type=jnp.float32)
    t = jnp.maximum(t + b1_ref[...], 0.0)
    u = jnp.dot(t, w2_ref[...], preferred_element_type=jnp.float32)
    h2 = jnp.maximum(u + b2_ref[...], 0.0)
    y = jnp.dot(h1, fca_ref[...], preferred_element_type=jnp.float32)
    y = y + jnp.dot(h2, fcc_ref[...], preferred_element_type=jnp.float32)
    y_ref[...] = y + fcb_ref[0, 0]


def _tc_gin_head(h1, aggs, eps, W1, b1, W2, b2, fc_W, fc_b):
    return pl.pallas_call(
        _head_body,
        grid=(N // BM,),
        in_specs=[
            pl.BlockSpec(memory_space=pltpu.SMEM),
            pl.BlockSpec(memory_space=pltpu.SMEM),
            pl.BlockSpec((BM, D), lambda i: (i, 0)),
            pl.BlockSpec((NC, BM, D), lambda i: (0, i, 0)),
            pl.BlockSpec((D, D), lambda i: (0, 0)),
            pl.BlockSpec((1, D), lambda i: (0, 0)),
            pl.BlockSpec((D, D), lambda i: (0, 0)),
            pl.BlockSpec((1, D), lambda i: (0, 0)),
            pl.BlockSpec((D, 1), lambda i: (0, 0)),
            pl.BlockSpec((D, 1), lambda i: (0, 0)),
        ],
        out_specs=pl.BlockSpec((BM, 1), lambda i: (i, 0)),
        out_shape=jax.ShapeDtypeStruct((N, 1), jnp.float32),
    )(eps.reshape(1, 1), fc_b.reshape(1, 1), h1, aggs, W1, b1.reshape(1, D),
      W2, b2.reshape(1, D), fc_W[:D], fc_W[D:])


def kernel(x, edge_index, ano_label, W1_0, b1_0, W2_0, b2_0, eps_0,
           W1_1, b1_1, W2_1, b2_1, eps_1, fc_W, fc_b):
    del ano_label
    src = edge_index[0].astype(jnp.int32)
    dst = edge_index[1].astype(jnp.int32)
    pad = EPAD - E
    src_p = jnp.concatenate([src, jnp.zeros((pad,), jnp.int32)])
    dst_p = jnp.concatenate([dst, jnp.full((pad,), N, jnp.int32)])
    srcs = src_p.reshape(NT, CH, CHUNK)
    dsts = dst_p.reshape(NT, CH, CHUNK)
    zeros = jnp.zeros((RPT, D), jnp.bfloat16)

    aggs0 = _sc_segment_sum(x.astype(jnp.bfloat16), srcs, dsts, zeros)
    h1, h1_bf = _tc_gin_mlp(x, aggs0, eps_0, W1_0, b1_0, W2_0, b2_0)
    aggs1 = _sc_segment_sum(h1_bf, srcs, dsts, zeros)
    return _tc_gin_head(h1, aggs1, eps_1, W1_1, b1_1, W2_1, b2_1, fc_W, fc_b)
